# baseline (speedup 1.0000x reference)
import functools

import numpy as np
import jax
import jax.numpy as jnp
from jax.experimental import pallas as pl

_PARAM_RAW_DIM = (7, 9, 11, 14, 17)
_MAX_RAW = 17
_PARAM_DIM = 32
_N_TYPES = 5
_EMB_DIM = 16
_LEVELS = (8, 5, 5, 5)

_BBLK = 2048
_IDX_OFFSET = 500


def _fused(st_ref, p_ref, wpe_ref, w1a_ref, tb1_ref, w2_ref, b2_ref,
           w3_ref, b3_ref, w4_ref, b4_ref, wfi_ref, bfi_ref, wfo_ref, bfo_ref,
           wcls_ref, bcls_ref, wcld_ref, bcld_ref, wd1_ref, tbd1_ref,
           wd2_ref, bd2_ref, wd3_ref, bd3_ref, wraw_ref, braw_ref, mstack_ref,
           hl_ref, off_ref, basis_ref,
           padded_ref, mask_ref, cls_ref, closed_ref, zq_ref, idx_ref):
    f32 = jnp.float32
    dot = functools.partial(jnp.dot, preferred_element_type=f32)

    st = st_ref[...]
    onehot = (st == jax.lax.broadcasted_iota(jnp.int32, (1, _N_TYPES), 1)
              ).astype(f32)

    p = p_ref[...]
    peall = dot(p, wpe_ref[...])
    pe = onehot[:, 0:1] * peall[:, 0:_PARAM_DIM]
    for t in range(1, _N_TYPES):
        pe = pe + onehot[:, t:t + 1] * peall[:, t * _PARAM_DIM:(t + 1) * _PARAM_DIM]

    h = jnp.maximum(dot(pe, w1a_ref[...]) + dot(onehot, tb1_ref[...]), 0.0)
    h = jnp.maximum(dot(h, w2_ref[...]) + b2_ref[...], 0.0)
    h = jnp.maximum(dot(h, w3_ref[...]) + b3_ref[...], 0.0)
    z = dot(h, w4_ref[...]) + b4_ref[...]

    zf = dot(z, wfi_ref[...]) + bfi_ref[...]
    bounded = jnp.tanh(zf) * hl_ref[...] - off_ref[...]
    rounded = jnp.round(bounded)
    idx = jnp.round(dot(rounded, basis_ref[...]))
    idx_ref[...] = idx.astype(jnp.int32) + _IDX_OFFSET

    zq = dot(rounded, wfo_ref[...]) + bfo_ref[...]
    zq_ref[...] = zq

    cls_ref[...] = dot(zq, wcls_ref[...]) + bcls_ref[...]
    closed_ref[...] = dot(zq, wcld_ref[...]) + bcld_ref[...]

    hd = jnp.maximum(dot(zq, wd1_ref[...]) + dot(onehot, tbd1_ref[...]), 0.0)
    hd = jnp.maximum(dot(hd, wd2_ref[...]) + bd2_ref[...], 0.0)
    pdec = dot(hd, wd3_ref[...]) + bd3_ref[...]

    outs = dot(pdec, wraw_ref[...]) + braw_ref[...]
    padded = onehot[:, 0:1] * outs[:, 0:_MAX_RAW]
    for t in range(1, _N_TYPES):
        padded = padded + onehot[:, t:t + 1] * outs[:, t * _MAX_RAW:(t + 1) * _MAX_RAW]
    padded_ref[...] = padded
    mask_ref[...] = dot(onehot, mstack_ref[...]) > 0.5


def kernel(params, surface_type, type_emb, pe_params, enc_params, fsq_in,
           fsq_out, dec_params, cls_params, closed_params, raw_dec_params):
    B = params.shape[0]
    f32 = jnp.float32

    wpe_cols = []
    for t in range(_N_TYPES):
        W, _ = pe_params[t]
        d = _PARAM_RAW_DIM[t]
        wpe_cols.append(jnp.zeros((_MAX_RAW, _PARAM_DIM), f32).at[:d].set(W.T))
    wpe = jnp.concatenate(wpe_cols, axis=1)
    bpe = jnp.stack([pe_params[t][1] for t in range(_N_TYPES)])

    W1, b1 = enc_params[0]
    w1a = W1[:, :_PARAM_DIM].T
    tb1 = (b1[None, :] + type_emb @ W1[:, _PARAM_DIM:].T
           + bpe @ W1[:, :_PARAM_DIM].T)
    W2, b2 = enc_params[1]
    W3, b3 = enc_params[2]
    W4, b4 = enc_params[3]

    levels = np.array(_LEVELS, dtype=np.float64)
    half_l = ((levels - 1.0) * (1.0 + 1e-3) / 2.0).astype(np.float32)
    offset = np.where(levels % 2 == 0, 0.5, 0.0).astype(np.float32)
    shift = np.arctanh(offset / half_l.astype(np.float64)).astype(np.float32)
    half_width = np.array([l // 2 for l in _LEVELS], dtype=np.float32)
    basis = np.concatenate([[1], np.cumprod(_LEVELS[:-1])]).astype(np.float32)

    Wfi, bfi = fsq_in
    bfi = bfi + shift
    Wfo, bfo = fsq_out
    wfo = Wfo.T / half_width[:, None]
    hl = jnp.asarray(half_l[None, :])
    off = jnp.asarray(offset[None, :])
    basis_col = jnp.asarray(basis[:, None])
    Wc, bc = cls_params
    Wcl, bcl = closed_params

    Wd1, bd1 = dec_params[0]
    wd1 = Wd1[:, :128].T
    tbd1 = bd1[None, :] + type_emb @ Wd1[:, 128:].T
    Wd2, bd2 = dec_params[1]
    Wd3, bd3 = dec_params[2]

    wraw_cols, braw_cols = [], []
    for t in range(_N_TYPES):
        W, b = raw_dec_params[t]
        d = _PARAM_RAW_DIM[t]
        wraw_cols.append(jnp.zeros((_PARAM_DIM, _MAX_RAW), f32).at[:, :d].set(W.T))
        braw_cols.append(jnp.zeros((_MAX_RAW,), f32).at[:d].set(b))
    wraw = jnp.concatenate(wraw_cols, axis=1)
    braw = jnp.concatenate(braw_cols)[None, :]

    mstack = jnp.asarray(
        np.arange(_MAX_RAW)[None, :] < np.array(_PARAM_RAW_DIM)[:, None],
        dtype=f32)

    st2 = surface_type.reshape(B, 1).astype(jnp.int32)

    row = lambda w: pl.BlockSpec((_BBLK, w), lambda i: (i, 0))
    full = lambda a: pl.BlockSpec(a.shape, lambda i: (0,) * a.ndim)

    weights = [wpe, w1a, tb1, W2.T, b2[None], W3.T, b3[None], W4.T, b4[None],
               Wfi.T, bfi[None], wfo, bfo[None], Wc.T, bc[None],
               Wcl.T, bcl[None], wd1, tbd1, Wd2.T, bd2[None], Wd3.T, bd3[None],
               wraw, braw, mstack, hl, off, basis_col]

    out_shapes = (
        jax.ShapeDtypeStruct((B, _MAX_RAW), f32),
        jax.ShapeDtypeStruct((B, _MAX_RAW), jnp.bool_),
        jax.ShapeDtypeStruct((B, _N_TYPES), f32),
        jax.ShapeDtypeStruct((B, 2), f32),
        jax.ShapeDtypeStruct((B, 128), f32),
        jax.ShapeDtypeStruct((B, 1), jnp.int32),
    )
    out_specs = (row(_MAX_RAW), row(_MAX_RAW), row(_N_TYPES), row(2),
                 row(128), row(1))

    outs = pl.pallas_call(
        _fused,
        grid=(B // _BBLK,),
        in_specs=[row(1), row(_MAX_RAW)] + [full(w) for w in weights],
        out_specs=out_specs,
        out_shape=out_shapes,
    )(st2, params, *weights)

    padded, mask, cls, closed, zq, idx = outs
    return (padded, mask, cls, closed, zq, idx.reshape(B))


# speedup vs baseline: 1.1027x; 1.1027x over previous
import functools

import numpy as np
import jax
import jax.numpy as jnp
from jax.experimental import pallas as pl

_PARAM_RAW_DIM = (7, 9, 11, 14, 17)
_MAX_RAW = 17
_PARAM_DIM = 32
_N_TYPES = 5
_EMB_DIM = 16
_LEVELS = (8, 5, 5, 5)

_BBLK = 2048
_IDX_OFFSET = 500

_DNT = (((1,), (1,)), ((), ()))


def _dott(x, w):
    return jax.lax.dot_general(x, w, _DNT, preferred_element_type=jnp.float32)


def _fused(st_ref, p_ref, wpe_ref, bpe_ref, temb_ref, w1_ref, b1_ref,
           w2_ref, b2_ref, w3_ref, b3_ref, w4_ref, b4_ref,
           wfi_ref, bfi_ref, wfo_ref, bfo_ref,
           wcls_ref, bcls_ref, wcld_ref, bcld_ref,
           wd1_ref, bd1_ref, wd2_ref, bd2_ref, wd3_ref, bd3_ref,
           wraw_ref, braw_ref, mstack_ref, hl_ref, off_ref, inv_hw_ref,
           basis_ref,
           padded_ref, mask_ref, cls_ref, closed_ref, zq_ref, idx_ref):
    f32 = jnp.float32
    dot = functools.partial(jnp.dot, preferred_element_type=f32)

    st = st_ref[...]
    onehot = (st == jax.lax.broadcasted_iota(jnp.int32, (1, _N_TYPES), 1)
              ).astype(f32)

    p = p_ref[...]
    peall = _dott(p, wpe_ref[...])
    pe = onehot[:, 0:1] * peall[:, 0:_PARAM_DIM]
    for t in range(1, _N_TYPES):
        pe = pe + onehot[:, t:t + 1] * peall[:, t * _PARAM_DIM:(t + 1) * _PARAM_DIM]

    w1 = w1_ref[...]
    temb = temb_ref[...]
    tb1 = (b1_ref[...] + _dott(temb, w1[:, _PARAM_DIM:])
           + _dott(bpe_ref[...], w1[:, :_PARAM_DIM]))
    h = jnp.maximum(_dott(pe, w1[:, :_PARAM_DIM]) + dot(onehot, tb1), 0.0)
    h = jnp.maximum(_dott(h, w2_ref[...]) + b2_ref[...], 0.0)
    h = jnp.maximum(_dott(h, w3_ref[...]) + b3_ref[...], 0.0)
    z = _dott(h, w4_ref[...]) + b4_ref[...]

    zf = _dott(z, wfi_ref[...]) + bfi_ref[...]
    bounded = jnp.tanh(zf) * hl_ref[...] - off_ref[...]
    rounded = jnp.round(bounded)
    idx = jnp.round(dot(rounded, basis_ref[...]))
    idx_ref[...] = idx.astype(jnp.int32) + _IDX_OFFSET

    codes = rounded * inv_hw_ref[...]
    zq = _dott(codes, wfo_ref[...]) + bfo_ref[...]
    zq_ref[...] = zq

    cls_ref[...] = _dott(zq, wcls_ref[...]) + bcls_ref[...]
    closed_ref[...] = _dott(zq, wcld_ref[...]) + bcld_ref[...]

    wd1 = wd1_ref[...]
    tbd1 = bd1_ref[...] + _dott(temb, wd1[:, 128:])
    hd = jnp.maximum(_dott(zq, wd1[:, :128]) + dot(onehot, tbd1), 0.0)
    hd = jnp.maximum(_dott(hd, wd2_ref[...]) + bd2_ref[...], 0.0)
    pdec = _dott(hd, wd3_ref[...]) + bd3_ref[...]

    outs = _dott(pdec, wraw_ref[...]) + braw_ref[...]
    padded = onehot[:, 0:1] * outs[:, 0:_MAX_RAW]
    for t in range(1, _N_TYPES):
        padded = padded + onehot[:, t:t + 1] * outs[:, t * _MAX_RAW:(t + 1) * _MAX_RAW]
    padded_ref[...] = padded
    mask_ref[...] = dot(onehot, mstack_ref[...]) > 0.5


def kernel(params, surface_type, type_emb, pe_params, enc_params, fsq_in,
           fsq_out, dec_params, cls_params, closed_params, raw_dec_params):
    B = params.shape[0]
    f32 = jnp.float32

    wpe = jnp.concatenate([
        jnp.pad(pe_params[t][0], ((0, 0), (0, _MAX_RAW - _PARAM_RAW_DIM[t])))
        for t in range(_N_TYPES)], axis=0)
    bpe = jnp.stack([pe_params[t][1] for t in range(_N_TYPES)])

    (W1, b1), (W2, b2), (W3, b3), (W4, b4) = enc_params

    levels = np.array(_LEVELS, dtype=np.float64)
    half_l = ((levels - 1.0) * (1.0 + 1e-3) / 2.0).astype(np.float32)
    offset = np.where(levels % 2 == 0, 0.5, 0.0).astype(np.float32)
    shift = np.arctanh(offset / half_l.astype(np.float64)).astype(np.float32)
    half_width = np.array([l // 2 for l in _LEVELS], dtype=np.float32)
    basis = np.concatenate([[1], np.cumprod(_LEVELS[:-1])]).astype(np.float32)

    Wfi, bfi = fsq_in
    bfi = bfi + shift
    Wfo, bfo = fsq_out
    Wc, bc = cls_params
    Wcl, bcl = closed_params
    (Wd1, bd1), (Wd2, bd2), (Wd3, bd3) = dec_params

    wraw = jnp.concatenate([
        jnp.pad(raw_dec_params[t][0],
                ((0, _MAX_RAW - _PARAM_RAW_DIM[t]), (0, 0)))
        for t in range(_N_TYPES)], axis=0)
    braw = jnp.concatenate([
        jnp.pad(raw_dec_params[t][1], (0, _MAX_RAW - _PARAM_RAW_DIM[t]))
        for t in range(_N_TYPES)])[None, :]

    mstack = jnp.asarray(
        np.arange(_MAX_RAW)[None, :] < np.array(_PARAM_RAW_DIM)[:, None],
        dtype=f32)
    hl = jnp.asarray(half_l[None, :])
    off = jnp.asarray(offset[None, :])
    inv_hw = jnp.asarray((1.0 / half_width)[None, :])
    basis_col = jnp.asarray(basis[:, None])

    st2 = surface_type.reshape(B, 1).astype(jnp.int32)

    row = lambda w: pl.BlockSpec((_BBLK, w), lambda i: (i, 0))
    full = lambda a: pl.BlockSpec(a.shape, lambda i: (0,) * a.ndim)

    weights = [wpe, bpe, type_emb, W1, b1[None], W2, b2[None], W3, b3[None],
               W4, b4[None], Wfi, bfi[None], Wfo, bfo[None], Wc, bc[None],
               Wcl, bcl[None], Wd1, bd1[None], Wd2, bd2[None], Wd3, bd3[None],
               wraw, braw, mstack, hl, off, inv_hw, basis_col]

    out_shapes = (
        jax.ShapeDtypeStruct((B, _MAX_RAW), f32),
        jax.ShapeDtypeStruct((B, _MAX_RAW), jnp.bool_),
        jax.ShapeDtypeStruct((B, _N_TYPES), f32),
        jax.ShapeDtypeStruct((B, 2), f32),
        jax.ShapeDtypeStruct((B, 128), f32),
        jax.ShapeDtypeStruct((B, 1), jnp.int32),
    )
    out_specs = (row(_MAX_RAW), row(_MAX_RAW), row(_N_TYPES), row(2),
                 row(128), row(1))

    outs = pl.pallas_call(
        _fused,
        grid=(B // _BBLK,),
        in_specs=[row(1), row(_MAX_RAW)] + [full(w) for w in weights],
        out_specs=out_specs,
        out_shape=out_shapes,
    )(st2, params, *weights)

    padded, mask, cls, closed, zq, idx = outs
    return (padded, mask, cls, closed, zq, idx.reshape(B))


# speedup vs baseline: 1.2212x; 1.1075x over previous
import functools

import numpy as np
import jax
import jax.numpy as jnp
from jax.experimental import pallas as pl

_PARAM_RAW_DIM = (7, 9, 11, 14, 17)
_MAX_RAW = 17
_PARAM_DIM = 32
_N_TYPES = 5
_EMB_DIM = 16
_LEVELS = (8, 5, 5, 5)

_BBLK = 2048
_IDX_OFFSET = 500

_DNT = (((1,), (1,)), ((), ()))


def _dott(x, w):
    return jax.lax.dot_general(x, w, _DNT, preferred_element_type=jnp.float32)


def _fused(st_ref, p_ref,
           wpe0_ref, bpe0_ref, wpe1_ref, bpe1_ref, wpe2_ref, bpe2_ref,
           wpe3_ref, bpe3_ref, wpe4_ref, bpe4_ref,
           temb_ref, w1_ref, b1_ref, w2_ref, b2_ref, w3_ref, b3_ref,
           w4_ref, b4_ref, wfi_ref, bfi_ref, wfo_ref, bfo_ref,
           wcls_ref, bcls_ref, wcld_ref, bcld_ref,
           wd1_ref, bd1_ref, wd2_ref, bd2_ref, wd3_ref, bd3_ref,
           wr0_ref, br0_ref, wr1_ref, br1_ref, wr2_ref, br2_ref,
           wr3_ref, br3_ref, wr4_ref, br4_ref,
           mstack_ref, shift_ref, hl_ref, off_ref, inv_hw_ref, basis_ref,
           padded_ref, mask_ref, cls_ref, closed_ref, zq_ref, idx_ref):
    f32 = jnp.float32
    dot = functools.partial(jnp.dot, preferred_element_type=f32)

    st = st_ref[...]
    onehot = (st == jax.lax.broadcasted_iota(jnp.int32, (1, _N_TYPES), 1)
              ).astype(f32)

    wpe_refs = (wpe0_ref, wpe1_ref, wpe2_ref, wpe3_ref, wpe4_ref)
    bpe_refs = (bpe0_ref, bpe1_ref, bpe2_ref, bpe3_ref, bpe4_ref)
    wpe = jnp.concatenate(
        [jnp.pad(r[...], ((0, 0), (0, _MAX_RAW - _PARAM_RAW_DIM[t])))
         for t, r in enumerate(wpe_refs)], axis=0)
    bpe = jnp.concatenate([r[...] for r in bpe_refs], axis=0)

    wr_refs = (wr0_ref, wr1_ref, wr2_ref, wr3_ref, wr4_ref)
    br_refs = (br0_ref, br1_ref, br2_ref, br3_ref, br4_ref)
    wraw = jnp.concatenate(
        [jnp.pad(r[...], ((0, _MAX_RAW - _PARAM_RAW_DIM[t]), (0, 0)))
         for t, r in enumerate(wr_refs)], axis=0)
    braw = jnp.concatenate(
        [jnp.pad(r[...], ((0, 0), (0, _MAX_RAW - _PARAM_RAW_DIM[t])))
         for t, r in enumerate(br_refs)], axis=1)

    p = p_ref[...]
    peall = _dott(p, wpe)
    pe = onehot[:, 0:1] * peall[:, 0:_PARAM_DIM]
    for t in range(1, _N_TYPES):
        pe = pe + onehot[:, t:t + 1] * peall[:, t * _PARAM_DIM:(t + 1) * _PARAM_DIM]

    w1 = w1_ref[...]
    temb = temb_ref[...]
    tb1 = (b1_ref[...] + _dott(temb, w1[:, _PARAM_DIM:])
           + _dott(bpe, w1[:, :_PARAM_DIM]))
    h = jnp.maximum(_dott(pe, w1[:, :_PARAM_DIM]) + dot(onehot, tb1), 0.0)
    h = jnp.maximum(_dott(h, w2_ref[...]) + b2_ref[...], 0.0)
    h = jnp.maximum(_dott(h, w3_ref[...]) + b3_ref[...], 0.0)
    z = _dott(h, w4_ref[...]) + b4_ref[...]

    zf = _dott(z, wfi_ref[...]) + bfi_ref[...] + shift_ref[...]
    bounded = jnp.tanh(zf) * hl_ref[...] - off_ref[...]
    rounded = jnp.round(bounded)
    idx = jnp.round(dot(rounded, basis_ref[...]))
    idx_ref[...] = idx.astype(jnp.int32) + _IDX_OFFSET

    codes = rounded * inv_hw_ref[...]
    zq = _dott(codes, wfo_ref[...]) + bfo_ref[...]
    zq_ref[...] = zq

    cls_ref[...] = _dott(zq, wcls_ref[...]) + bcls_ref[...]
    closed_ref[...] = _dott(zq, wcld_ref[...]) + bcld_ref[...]

    wd1 = wd1_ref[...]
    tbd1 = bd1_ref[...] + _dott(temb, wd1[:, 128:])
    hd = jnp.maximum(_dott(zq, wd1[:, :128]) + dot(onehot, tbd1), 0.0)
    hd = jnp.maximum(_dott(hd, wd2_ref[...]) + bd2_ref[...], 0.0)
    pdec = _dott(hd, wd3_ref[...]) + bd3_ref[...]

    outs = _dott(pdec, wraw) + braw
    padded = onehot[:, 0:1] * outs[:, 0:_MAX_RAW]
    for t in range(1, _N_TYPES):
        padded = padded + onehot[:, t:t + 1] * outs[:, t * _MAX_RAW:(t + 1) * _MAX_RAW]
    padded_ref[...] = padded
    mask_ref[...] = dot(onehot, mstack_ref[...]) > 0.5


def kernel(params, surface_type, type_emb, pe_params, enc_params, fsq_in,
           fsq_out, dec_params, cls_params, closed_params, raw_dec_params):
    B = params.shape[0]
    f32 = jnp.float32

    levels = np.array(_LEVELS, dtype=np.float64)
    half_l = ((levels - 1.0) * (1.0 + 1e-3) / 2.0).astype(np.float32)
    offset = np.where(levels % 2 == 0, 0.5, 0.0).astype(np.float32)
    shift = np.arctanh(offset / half_l.astype(np.float64)).astype(np.float32)
    half_width = np.array([l // 2 for l in _LEVELS], dtype=np.float32)
    basis = np.concatenate([[1], np.cumprod(_LEVELS[:-1])]).astype(np.float32)

    (W1, b1), (W2, b2), (W3, b3), (W4, b4) = enc_params
    Wfi, bfi = fsq_in
    Wfo, bfo = fsq_out
    Wc, bc = cls_params
    Wcl, bcl = closed_params
    (Wd1, bd1), (Wd2, bd2), (Wd3, bd3) = dec_params

    mstack = jnp.asarray(
        np.arange(_MAX_RAW)[None, :] < np.array(_PARAM_RAW_DIM)[:, None],
        dtype=f32)
    shift_in = jnp.asarray(shift[None, :])
    hl = jnp.asarray(half_l[None, :])
    off = jnp.asarray(offset[None, :])
    inv_hw = jnp.asarray((1.0 / half_width)[None, :])
    basis_col = jnp.asarray(basis[:, None])

    st2 = surface_type.reshape(B, 1)

    inputs = [st2, params]
    for t in range(_N_TYPES):
        inputs += [pe_params[t][0], pe_params[t][1].reshape(1, _PARAM_DIM)]
    inputs += [type_emb, W1, b1.reshape(1, -1), W2, b2.reshape(1, -1),
               W3, b3.reshape(1, -1), W4, b4.reshape(1, -1),
               Wfi, bfi.reshape(1, -1), Wfo, bfo.reshape(1, -1),
               Wc, bc.reshape(1, -1), Wcl, bcl.reshape(1, -1),
               Wd1, bd1.reshape(1, -1), Wd2, bd2.reshape(1, -1),
               Wd3, bd3.reshape(1, -1)]
    for t in range(_N_TYPES):
        inputs += [raw_dec_params[t][0], raw_dec_params[t][1].reshape(1, -1)]
    inputs += [mstack, shift_in, hl, off, inv_hw, basis_col]

    row = lambda w: pl.BlockSpec((_BBLK, w), lambda i: (i, 0))
    full = lambda a: pl.BlockSpec(a.shape, lambda i: (0,) * a.ndim)

    out_shapes = (
        jax.ShapeDtypeStruct((B, _MAX_RAW), f32),
        jax.ShapeDtypeStruct((B, _MAX_RAW), jnp.bool_),
        jax.ShapeDtypeStruct((B, _N_TYPES), f32),
        jax.ShapeDtypeStruct((B, 2), f32),
        jax.ShapeDtypeStruct((B, 128), f32),
        jax.ShapeDtypeStruct((B, 1), jnp.int32),
    )
    out_specs = (row(_MAX_RAW), row(_MAX_RAW), row(_N_TYPES), row(2),
                 row(128), row(1))

    outs = pl.pallas_call(
        _fused,
        grid=(B // _BBLK,),
        in_specs=[row(1), row(_MAX_RAW)] + [full(a) for a in inputs[2:]],
        out_specs=out_specs,
        out_shape=out_shapes,
    )(*inputs)

    padded, mask, cls, closed, zq, idx = outs
    return (padded, mask, cls, closed, zq, idx.reshape(B))
